# baseline (speedup 1.0000x reference)
import functools

import jax
import jax.numpy as jnp
from jax import lax
from jax.experimental import pallas as pl
from jax.experimental.pallas import tpu as pltpu
from jax.experimental.pallas import tpu_sc as plsc

N = 10000
E = 320000
D = 128

NC = 2
NS = 16

NPAD = 10240
ROWS_PER_TILE = NPAD // NS
CHUNK = 128
CHUNKS_PER_WORKER = 80
E_PAD = NC * NS * CHUNKS_PER_WORKER * CHUNK

_mesh = plsc.VectorSubcoreMesh(
    core_axis_name="c", subcore_axis_name="s", num_cores=NC, num_subcores=NS
)


@functools.partial(
    pl.kernel,
    out_type=jax.ShapeDtypeStruct((NC, NPAD), jnp.float32),
    mesh=_mesh,
    scratch_types=[
        pltpu.VMEM((CHUNKS_PER_WORKER, CHUNK), jnp.int32),
        pltpu.VMEM((CHUNK,), jnp.float32),
        pltpu.VMEM((ROWS_PER_TILE,), jnp.float32),
        pltpu.VMEM_SHARED((NPAD,), jnp.float32),
        pltpu.SemaphoreType.DMA,
    ],
)
def _deg_kernel(eidx_hbm, deg_out, idx_v, ones_v, zrow_v, hist_sh, sem_s):
    c = lax.axis_index("c")
    s = lax.axis_index("s")
    w = s * NC + c
    for i in range(CHUNK // 16):
        ones_v[pl.ds(i * 16, 16)] = jnp.full((16,), 1.0, jnp.float32)
    for i in range(ROWS_PER_TILE // 16):
        zrow_v[pl.ds(i * 16, 16)] = jnp.zeros((16,), jnp.float32)
    pltpu.sync_copy(zrow_v, hist_sh.at[pl.ds(s * ROWS_PER_TILE, ROWS_PER_TILE)])
    pltpu.sync_copy(
        eidx_hbm.at[1, pl.ds(w * CHUNKS_PER_WORKER, CHUNKS_PER_WORKER)], idx_v)
    plsc.subcore_barrier()

    DEPTH = 8

    def body(j, carry):
        pltpu.async_copy(ones_v, hist_sh.at[idx_v.at[j]], sem_s, add=True)

        @pl.when(j >= DEPTH)
        def _():
            pltpu.make_async_copy(ones_v, hist_sh.at[idx_v.at[j]], sem_s).wait()

        return carry

    lax.fori_loop(0, CHUNKS_PER_WORKER, body, 0)
    for _ in range(DEPTH):
        pltpu.make_async_copy(ones_v, hist_sh.at[idx_v.at[0]], sem_s).wait()
    plsc.subcore_barrier()
    pltpu.sync_copy(
        hist_sh.at[pl.ds(s * ROWS_PER_TILE, ROWS_PER_TILE)],
        deg_out.at[c, pl.ds(s * ROWS_PER_TILE, ROWS_PER_TILE)],
    )


@functools.partial(
    pl.kernel,
    out_type=jax.ShapeDtypeStruct((NC * NPAD, D), jnp.float32),
    mesh=_mesh,
    scratch_types=[
        pltpu.VMEM((2, CHUNK), jnp.int32),
        pltpu.VMEM((2, CHUNK), jnp.int32),
        pltpu.VMEM((2, CHUNK), jnp.int32),
        pltpu.VMEM((2, CHUNK), jnp.int32),
        pltpu.VMEM((CHUNK, D), jnp.float32),
        pltpu.VMEM((CHUNK, D), jnp.float32),
        pltpu.VMEM_SHARED((NPAD, D), jnp.float32),
        pltpu.SemaphoreType.DMA,
        pltpu.SemaphoreType.DMA,
        pltpu.SemaphoreType.DMA,
    ],
)
def _prop_kernel(y_hbm, eidx_hbm, z_out,
                 idx_a, idx_b, idx_c, idx_d, rows_a, rows_b, acc_sh,
                 sem_i, sem_g, sem_s):
    idx = [idx_a, idx_b, idx_c, idx_d]
    rows = [rows_a, rows_b]
    c = lax.axis_index("c")
    s = lax.axis_index("s")
    w = s * NC + c
    r0 = s * ROWS_PER_TILE

    init_d = [
        pltpu.async_copy(
            y_hbm.at[pl.ds(r0 + q * (ROWS_PER_TILE // 4), ROWS_PER_TILE // 4)],
            acc_sh.at[pl.ds(r0 + q * (ROWS_PER_TILE // 4), ROWS_PER_TILE // 4)],
            sem_s)
        for q in range(4)
    ]
    for d in init_d:
        d.wait()
    plsc.subcore_barrier()

    def idx_fetch(j, p):
        row = w * CHUNKS_PER_WORKER + j
        pltpu.async_copy(eidx_hbm.at[0, row], idx[p].at[0], sem_i)
        pltpu.async_copy(eidx_hbm.at[1, row], idx[p].at[1], sem_i)

    def idx_wait(p):
        pltpu.make_async_copy(eidx_hbm.at[0, 0], idx[p].at[0], sem_i).wait()
        pltpu.make_async_copy(eidx_hbm.at[1, 0], idx[p].at[1], sem_i).wait()

    def gather(rp, ip):
        pltpu.async_copy(y_hbm.at[idx[ip].at[0]], rows[rp], sem_g)

    def gather_wait(p):
        pltpu.make_async_copy(y_hbm.at[idx[0].at[0]], rows[p], sem_g).wait()

    def scatter_fire(rp, ip):
        pltpu.async_copy(rows[rp], acc_sh.at[idx[ip].at[1]], sem_s, add=True)

    def scatter_wait():
        pltpu.make_async_copy(rows[0], acc_sh.at[idx[0].at[1]], sem_s).wait()

    idx_fetch(0, 0)
    idx_fetch(1, 1)

    UNROLL = 4

    def body(g, carry):
        for u in range(UNROLL):
            j = g * UNROLL + u
            rp = u % 2
            ip = u % 4

            @pl.when(j < CHUNKS_PER_WORKER)
            def _():
                @pl.when(j >= 2)
                def _():
                    scatter_wait()

                idx_wait(ip)
                gather(rp, ip)

                @pl.when(j >= 1)
                def _():
                    gather_wait(1 - rp)
                    scatter_fire(1 - rp, (ip + 3) % 4)

                @pl.when(j + 2 < CHUNKS_PER_WORKER)
                def _():
                    idx_fetch(j + 2, (ip + 2) % 4)

        return carry

    lax.fori_loop(0, (CHUNKS_PER_WORKER + UNROLL - 1) // UNROLL, body, 0)
    scatter_wait()
    gather_wait((CHUNKS_PER_WORKER - 1) % 2)
    scatter_fire((CHUNKS_PER_WORKER - 1) % 2, (CHUNKS_PER_WORKER - 1) % 4)
    scatter_wait()
    plsc.subcore_barrier()
    out_d = [
        pltpu.async_copy(
            acc_sh.at[pl.ds(r0 + q * (ROWS_PER_TILE // 4), ROWS_PER_TILE // 4)],
            z_out.at[pl.ds(c * NPAD + r0 + q * (ROWS_PER_TILE // 4),
                           ROWS_PER_TILE // 4)],
            sem_s)
        for q in range(4)
    ]
    for d in out_d:
        d.wait()


_BLK = 1024
_GRID = NPAD // _BLK


def _dinv_of(deg_ref):
    return lax.rsqrt(deg_ref[0] + deg_ref[1] + 1.0)


def _tc_mm_body(x_ref, w_ref, y_ref):
    y_ref[...] = jnp.dot(x_ref[...], w_ref[...],
                         preferred_element_type=jnp.float32)


def _tc_scale_body(xw_ref, deg_ref, y_ref):
    y_ref[...] = xw_ref[...] * _dinv_of(deg_ref)


def _tc_mid_body(z_ref, y_ref, w_ref, b_ref, deg_ref, o_ref):
    dinv = _dinv_of(deg_ref)
    z = z_ref[0] + z_ref[1] - y_ref[...]
    h = jnp.maximum(z * dinv + b_ref[...], 0.0)
    o_ref[...] = (
        jnp.dot(h, w_ref[...], preferred_element_type=jnp.float32) * dinv
    )


def _tc3_body(z_ref, y_ref, w_ref, b_ref, blin_ref, deg_ref, o_ref):
    dinv = _dinv_of(deg_ref)
    z = z_ref[0] + z_ref[1] - y_ref[...]
    h = jnp.maximum(z * dinv + b_ref[...], 0.0)
    o_ref[...] = (
        jnp.dot(h, w_ref[...], preferred_element_type=jnp.float32)
        + blin_ref[...]
    )


_deg_spec = pl.BlockSpec((NC, _BLK, 1), lambda i: (0, i, 0))
_z_spec = pl.BlockSpec((NC, _BLK, D), lambda i: (0, i, 0))
_row_spec = pl.BlockSpec((_BLK, D), lambda i: (i, 0))

_tc_mm = pl.pallas_call(
    _tc_mm_body,
    grid=(N // 1000,),
    in_specs=[
        pl.BlockSpec((1000, D), lambda i: (i, 0)),
        pl.BlockSpec((D, D), lambda i: (0, 0)),
    ],
    out_specs=pl.BlockSpec((1000, D), lambda i: (i, 0)),
    out_shape=jax.ShapeDtypeStruct((NPAD, D), jnp.float32),
)

_tc_scale = pl.pallas_call(
    _tc_scale_body,
    grid=(_GRID,),
    in_specs=[_row_spec, _deg_spec],
    out_specs=_row_spec,
    out_shape=jax.ShapeDtypeStruct((NPAD, D), jnp.float32),
)

_tc_mid = pl.pallas_call(
    _tc_mid_body,
    grid=(_GRID,),
    in_specs=[
        _z_spec,
        _row_spec,
        pl.BlockSpec((D, D), lambda i: (0, 0)),
        pl.BlockSpec((1, D), lambda i: (0, 0)),
        _deg_spec,
    ],
    out_specs=_row_spec,
    out_shape=jax.ShapeDtypeStruct((NPAD, D), jnp.float32),
)

_tc3 = pl.pallas_call(
    _tc3_body,
    grid=(N // 1000,),
    in_specs=[
        pl.BlockSpec((NC, 1000, D), lambda i: (0, i, 0)),
        pl.BlockSpec((1000, D), lambda i: (i, 0)),
        pl.BlockSpec((D, 1), lambda i: (0, 0)),
        pl.BlockSpec((1, D), lambda i: (0, 0)),
        pl.BlockSpec((1, 1), lambda i: (0, 0)),
        pl.BlockSpec((NC, 1000, 1), lambda i: (0, i, 0)),
    ],
    out_specs=pl.BlockSpec((1000, 1), lambda i: (i, 0)),
    out_shape=jax.ShapeDtypeStruct((N, 1), jnp.float32),
)


def kernel(x, edge_index, W1, b1, W2, b2, Wlin, blin):
    npad_e = E_PAD - E
    pad_idx = (jnp.arange(npad_e, dtype=jnp.int32) % (NPAD - N)) + N
    e_all = jnp.concatenate(
        [edge_index.astype(jnp.int32),
         jnp.broadcast_to(pad_idx, (2, npad_e))], axis=1)
    eidx = e_all.reshape(2, -1, CHUNK)

    deg = _deg_kernel(eidx)
    deg3 = deg.reshape(NC, NPAD, 1)
    xw1 = _tc_mm(x, W1)
    y1 = _tc_scale(xw1, deg3)
    z1 = _prop_kernel(y1, eidx).reshape(NC, NPAD, D)
    y2 = _tc_mid(z1, y1, W2, b1.reshape(1, D), deg3)
    z2 = _prop_kernel(y2, eidx).reshape(NC, NPAD, D)
    return _tc3(z2, y2, Wlin, b2.reshape(1, D), blin.reshape(1, 1), deg3)


# speedup vs baseline: 1.0182x; 1.0182x over previous
import functools

import jax
import jax.numpy as jnp
from jax import lax
from jax.experimental import pallas as pl
from jax.experimental.pallas import tpu as pltpu
from jax.experimental.pallas import tpu_sc as plsc

N = 10000
E = 320000
D = 128

NC = 2
NS = 16

NPAD = 10240
ROWS_PER_TILE = NPAD // NS
CHUNK = 128
CHUNKS_PER_WORKER = 80
E_PAD = NC * NS * CHUNKS_PER_WORKER * CHUNK

_mesh = plsc.VectorSubcoreMesh(
    core_axis_name="c", subcore_axis_name="s", num_cores=NC, num_subcores=NS
)


@functools.partial(
    pl.kernel,
    out_type=jax.ShapeDtypeStruct((NC, NPAD), jnp.float32),
    mesh=_mesh,
    scratch_types=[
        pltpu.VMEM((CHUNKS_PER_WORKER, 2, CHUNK), jnp.int32),
        pltpu.VMEM((CHUNK,), jnp.float32),
        pltpu.VMEM((ROWS_PER_TILE,), jnp.float32),
        pltpu.VMEM_SHARED((NPAD,), jnp.float32),
        pltpu.SemaphoreType.DMA,
    ],
)
def _deg_kernel(eidx_hbm, deg_out, idx_v, ones_v, zrow_v, hist_sh, sem_s):
    c = lax.axis_index("c")
    s = lax.axis_index("s")
    w = s * NC + c
    for i in range(CHUNK // 16):
        ones_v[pl.ds(i * 16, 16)] = jnp.full((16,), 1.0, jnp.float32)
    for i in range(ROWS_PER_TILE // 16):
        zrow_v[pl.ds(i * 16, 16)] = jnp.zeros((16,), jnp.float32)
    pltpu.sync_copy(zrow_v, hist_sh.at[pl.ds(s * ROWS_PER_TILE, ROWS_PER_TILE)])
    pltpu.sync_copy(eidx_hbm.at[pl.ds(w * CHUNKS_PER_WORKER, CHUNKS_PER_WORKER)],
                    idx_v)
    plsc.subcore_barrier()

    DEPTH = 8

    def body(j, carry):
        pltpu.async_copy(ones_v, hist_sh.at[idx_v.at[j, 1]], sem_s, add=True)

        @pl.when(j >= DEPTH)
        def _():
            pltpu.make_async_copy(ones_v, hist_sh.at[idx_v.at[j, 1]], sem_s).wait()

        return carry

    lax.fori_loop(0, CHUNKS_PER_WORKER, body, 0)
    for _ in range(DEPTH):
        pltpu.make_async_copy(ones_v, hist_sh.at[idx_v.at[0, 1]], sem_s).wait()
    plsc.subcore_barrier()
    pltpu.sync_copy(
        hist_sh.at[pl.ds(s * ROWS_PER_TILE, ROWS_PER_TILE)],
        deg_out.at[c, pl.ds(s * ROWS_PER_TILE, ROWS_PER_TILE)],
    )


@functools.partial(
    pl.kernel,
    out_type=jax.ShapeDtypeStruct((NC * NPAD, D), jnp.float32),
    mesh=_mesh,
    scratch_types=[
        pltpu.VMEM((2, CHUNK), jnp.int32),
        pltpu.VMEM((2, CHUNK), jnp.int32),
        pltpu.VMEM((2, CHUNK), jnp.int32),
        pltpu.VMEM((2, CHUNK), jnp.int32),
        pltpu.VMEM((CHUNK, D), jnp.float32),
        pltpu.VMEM((CHUNK, D), jnp.float32),
        pltpu.VMEM_SHARED((NPAD, D), jnp.float32),
        pltpu.SemaphoreType.DMA,
        pltpu.SemaphoreType.DMA,
        pltpu.SemaphoreType.DMA,
    ],
)
def _prop_kernel(y_hbm, eidx_hbm, z_out,
                 idx_a, idx_b, idx_c, idx_d, rows_a, rows_b, acc_sh,
                 sem_i, sem_g, sem_s):
    idx = [idx_a, idx_b, idx_c, idx_d]
    rows = [rows_a, rows_b]
    c = lax.axis_index("c")
    s = lax.axis_index("s")
    w = s * NC + c
    r0 = s * ROWS_PER_TILE

    init_d = [
        pltpu.async_copy(
            y_hbm.at[pl.ds(r0 + q * (ROWS_PER_TILE // 4), ROWS_PER_TILE // 4)],
            acc_sh.at[pl.ds(r0 + q * (ROWS_PER_TILE // 4), ROWS_PER_TILE // 4)],
            sem_s)
        for q in range(4)
    ]
    for d in init_d:
        d.wait()
    plsc.subcore_barrier()

    def idx_fetch(j, p):
        pltpu.async_copy(eidx_hbm.at[w * CHUNKS_PER_WORKER + j], idx[p], sem_i)

    def idx_wait(p):
        pltpu.make_async_copy(eidx_hbm.at[0], idx[p], sem_i).wait()

    def gather(rp, ip):
        pltpu.async_copy(y_hbm.at[idx[ip].at[0]], rows[rp], sem_g)

    def gather_wait(p):
        pltpu.make_async_copy(y_hbm.at[idx[0].at[0]], rows[p], sem_g).wait()

    def scatter_fire(rp, ip):
        pltpu.async_copy(rows[rp], acc_sh.at[idx[ip].at[1]], sem_s, add=True)

    def scatter_wait():
        pltpu.make_async_copy(rows[0], acc_sh.at[idx[0].at[1]], sem_s).wait()

    idx_fetch(0, 0)
    idx_fetch(1, 1)

    UNROLL = 4

    def body(g, carry):
        for u in range(UNROLL):
            j = g * UNROLL + u
            rp = u % 2
            ip = u % 4

            @pl.when(j < CHUNKS_PER_WORKER)
            def _():
                @pl.when(j >= 2)
                def _():
                    scatter_wait()

                idx_wait(ip)
                gather(rp, ip)

                @pl.when(j >= 1)
                def _():
                    gather_wait(1 - rp)
                    scatter_fire(1 - rp, (ip + 3) % 4)

                @pl.when(j + 2 < CHUNKS_PER_WORKER)
                def _():
                    idx_fetch(j + 2, (ip + 2) % 4)

        return carry

    lax.fori_loop(0, (CHUNKS_PER_WORKER + UNROLL - 1) // UNROLL, body, 0)
    scatter_wait()
    gather_wait((CHUNKS_PER_WORKER - 1) % 2)
    scatter_fire((CHUNKS_PER_WORKER - 1) % 2, (CHUNKS_PER_WORKER - 1) % 4)
    scatter_wait()
    plsc.subcore_barrier()
    out_d = [
        pltpu.async_copy(
            acc_sh.at[pl.ds(r0 + q * (ROWS_PER_TILE // 4), ROWS_PER_TILE // 4)],
            z_out.at[pl.ds(c * NPAD + r0 + q * (ROWS_PER_TILE // 4),
                           ROWS_PER_TILE // 4)],
            sem_s)
        for q in range(4)
    ]
    for d in out_d:
        d.wait()


_BLK = 1024
_GRID = NPAD // _BLK


def _dinv_of(deg_ref):
    return lax.rsqrt(deg_ref[0] + deg_ref[1] + 1.0)


def _tc_mm_body(x_ref, w_ref, y_ref):
    y_ref[...] = jnp.dot(x_ref[...], w_ref[...],
                         preferred_element_type=jnp.float32)


def _tc_scale_body(xw_ref, deg_ref, y_ref):
    y_ref[...] = xw_ref[...] * _dinv_of(deg_ref)


def _tc_mid_body(z_ref, y_ref, w_ref, b_ref, deg_ref, o_ref):
    dinv = _dinv_of(deg_ref)
    z = z_ref[0] + z_ref[1] - y_ref[...]
    h = jnp.maximum(z * dinv + b_ref[...], 0.0)
    o_ref[...] = (
        jnp.dot(h, w_ref[...], preferred_element_type=jnp.float32) * dinv
    )


def _tc3_body(z_ref, y_ref, w_ref, b_ref, blin_ref, deg_ref, o_ref):
    dinv = _dinv_of(deg_ref)
    z = z_ref[0] + z_ref[1] - y_ref[...]
    h = jnp.maximum(z * dinv + b_ref[...], 0.0)
    o_ref[...] = (
        jnp.dot(h, w_ref[...], preferred_element_type=jnp.float32)
        + blin_ref[...]
    )


_deg_spec = pl.BlockSpec((NC, _BLK, 1), lambda i: (0, i, 0))
_z_spec = pl.BlockSpec((NC, _BLK, D), lambda i: (0, i, 0))
_row_spec = pl.BlockSpec((_BLK, D), lambda i: (i, 0))

_tc_mm = pl.pallas_call(
    _tc_mm_body,
    grid=(N // 2000,),
    in_specs=[
        pl.BlockSpec((2000, D), lambda i: (i, 0)),
        pl.BlockSpec((D, D), lambda i: (0, 0)),
    ],
    out_specs=pl.BlockSpec((2000, D), lambda i: (i, 0)),
    out_shape=jax.ShapeDtypeStruct((NPAD, D), jnp.float32),
)

_tc_scale = pl.pallas_call(
    _tc_scale_body,
    grid=(_GRID,),
    in_specs=[_row_spec, _deg_spec],
    out_specs=_row_spec,
    out_shape=jax.ShapeDtypeStruct((NPAD, D), jnp.float32),
)

_tc_mid = pl.pallas_call(
    _tc_mid_body,
    grid=(_GRID,),
    in_specs=[
        _z_spec,
        _row_spec,
        pl.BlockSpec((D, D), lambda i: (0, 0)),
        pl.BlockSpec((1, D), lambda i: (0, 0)),
        _deg_spec,
    ],
    out_specs=_row_spec,
    out_shape=jax.ShapeDtypeStruct((NPAD, D), jnp.float32),
)

_tc3 = pl.pallas_call(
    _tc3_body,
    grid=(N // 2000,),
    in_specs=[
        pl.BlockSpec((NC, 2000, D), lambda i: (0, i, 0)),
        pl.BlockSpec((2000, D), lambda i: (i, 0)),
        pl.BlockSpec((D, 1), lambda i: (0, 0)),
        pl.BlockSpec((1, D), lambda i: (0, 0)),
        pl.BlockSpec((1, 1), lambda i: (0, 0)),
        pl.BlockSpec((NC, 2000, 1), lambda i: (0, i, 0)),
    ],
    out_specs=pl.BlockSpec((2000, 1), lambda i: (i, 0)),
    out_shape=jax.ShapeDtypeStruct((N, 1), jnp.float32),
)


def kernel(x, edge_index, W1, b1, W2, b2, Wlin, blin):
    npad_e = E_PAD - E
    pad_idx = (jnp.arange(npad_e, dtype=jnp.int32) % (NPAD - N)) + N
    e_all = jnp.concatenate(
        [edge_index.astype(jnp.int32),
         jnp.broadcast_to(pad_idx, (2, npad_e))], axis=1)
    eidx = jnp.transpose(e_all.reshape(2, -1, CHUNK), (1, 0, 2))

    deg = _deg_kernel(eidx)
    deg3 = deg.reshape(NC, NPAD, 1)
    xw1 = _tc_mm(x, W1)
    y1 = _tc_scale(xw1, deg3)
    z1 = _prop_kernel(y1, eidx).reshape(NC, NPAD, D)
    y2 = _tc_mid(z1, y1, W2, b1.reshape(1, D), deg3)
    z2 = _prop_kernel(y2, eidx).reshape(NC, NPAD, D)
    return _tc3(z2, y2, Wlin, b2.reshape(1, D), blin.reshape(1, 1), deg3)


# speedup vs baseline: 1.0339x; 1.0154x over previous
import functools

import jax
import jax.numpy as jnp
from jax import lax
from jax.experimental import pallas as pl
from jax.experimental.pallas import tpu as pltpu
from jax.experimental.pallas import tpu_sc as plsc

N = 10000
E = 320000
D = 128

NC = 2
NS = 16

NPAD = 10240
ROWS_PER_TILE = NPAD // NS
CHUNK = 128
CHUNKS_PER_WORKER = 80
E_PAD = NC * NS * CHUNKS_PER_WORKER * CHUNK

_mesh = plsc.VectorSubcoreMesh(
    core_axis_name="c", subcore_axis_name="s", num_cores=NC, num_subcores=NS
)


@functools.partial(
    pl.kernel,
    out_type=jax.ShapeDtypeStruct((NC, NPAD), jnp.float32),
    mesh=_mesh,
    scratch_types=[
        pltpu.VMEM((CHUNKS_PER_WORKER, 2, CHUNK), jnp.int32),
        pltpu.VMEM((CHUNK,), jnp.float32),
        pltpu.VMEM((ROWS_PER_TILE,), jnp.float32),
        pltpu.VMEM_SHARED((NPAD,), jnp.float32),
        pltpu.SemaphoreType.DMA,
    ],
)
def _deg_kernel(eidx_hbm, deg_out, idx_v, ones_v, zrow_v, hist_sh, sem_s):
    c = lax.axis_index("c")
    s = lax.axis_index("s")
    w = s * NC + c
    for i in range(CHUNK // 16):
        ones_v[pl.ds(i * 16, 16)] = jnp.full((16,), 1.0, jnp.float32)
    for i in range(ROWS_PER_TILE // 16):
        zrow_v[pl.ds(i * 16, 16)] = jnp.zeros((16,), jnp.float32)
    pltpu.sync_copy(zrow_v, hist_sh.at[pl.ds(s * ROWS_PER_TILE, ROWS_PER_TILE)])
    pltpu.sync_copy(eidx_hbm.at[pl.ds(w * CHUNKS_PER_WORKER, CHUNKS_PER_WORKER)],
                    idx_v)
    plsc.subcore_barrier()

    DEPTH = 8

    def body(j, carry):
        pltpu.async_copy(ones_v, hist_sh.at[idx_v.at[j, 1]], sem_s, add=True)

        @pl.when(j >= DEPTH)
        def _():
            pltpu.make_async_copy(ones_v, hist_sh.at[idx_v.at[j, 1]], sem_s).wait()

        return carry

    lax.fori_loop(0, CHUNKS_PER_WORKER, body, 0)
    for _ in range(DEPTH):
        pltpu.make_async_copy(ones_v, hist_sh.at[idx_v.at[0, 1]], sem_s).wait()
    plsc.subcore_barrier()
    pltpu.sync_copy(
        hist_sh.at[pl.ds(s * ROWS_PER_TILE, ROWS_PER_TILE)],
        deg_out.at[c, pl.ds(s * ROWS_PER_TILE, ROWS_PER_TILE)],
    )


@functools.partial(
    pl.kernel,
    out_type=jax.ShapeDtypeStruct((NC * NPAD, D), jnp.float32),
    mesh=_mesh,
    scratch_types=[
        pltpu.VMEM((2, CHUNK), jnp.int32),
        pltpu.VMEM((2, CHUNK), jnp.int32),
        pltpu.VMEM((2, CHUNK), jnp.int32),
        pltpu.VMEM((2, CHUNK), jnp.int32),
        pltpu.VMEM((CHUNK, D), jnp.float32),
        pltpu.VMEM((CHUNK, D), jnp.float32),
        pltpu.VMEM_SHARED((NPAD, D), jnp.float32),
        pltpu.SemaphoreType.DMA,
        pltpu.SemaphoreType.DMA,
        pltpu.SemaphoreType.DMA,
    ],
)
def _prop_kernel(y_hbm, eidx_hbm, z_out,
                 idx_a, idx_b, idx_c, idx_d, rows_a, rows_b, acc_sh,
                 sem_i, sem_g, sem_s):
    idx = [idx_a, idx_b, idx_c, idx_d]
    rows = [rows_a, rows_b]
    c = lax.axis_index("c")
    s = lax.axis_index("s")
    w = s * NC + c
    r0 = s * ROWS_PER_TILE

    init_d = [
        pltpu.async_copy(
            y_hbm.at[pl.ds(r0 + q * (ROWS_PER_TILE // 4), ROWS_PER_TILE // 4)],
            acc_sh.at[pl.ds(r0 + q * (ROWS_PER_TILE // 4), ROWS_PER_TILE // 4)],
            sem_s)
        for q in range(4)
    ]
    for d in init_d:
        d.wait()
    plsc.subcore_barrier()

    def idx_fetch(j, p):
        pltpu.async_copy(eidx_hbm.at[w * CHUNKS_PER_WORKER + j], idx[p], sem_i)

    def idx_wait(p):
        pltpu.make_async_copy(eidx_hbm.at[0], idx[p], sem_i).wait()

    def gather(rp, ip):
        pltpu.async_copy(y_hbm.at[idx[ip].at[0]], rows[rp], sem_g)

    def gather_wait(p):
        pltpu.make_async_copy(y_hbm.at[idx[0].at[0]], rows[p], sem_g).wait()

    def scatter_fire(rp, ip):
        pltpu.async_copy(rows[rp], acc_sh.at[idx[ip].at[1]], sem_s, add=True)

    def scatter_wait():
        pltpu.make_async_copy(rows[0], acc_sh.at[idx[0].at[1]], sem_s).wait()

    idx_fetch(0, 0)
    idx_fetch(1, 1)

    UNROLL = 4

    def body(g, carry):
        for u in range(UNROLL):
            j = g * UNROLL + u
            rp = u % 2
            ip = u % 4

            @pl.when(j < CHUNKS_PER_WORKER)
            def _():
                @pl.when(j >= 2)
                def _():
                    scatter_wait()

                idx_wait(ip)
                gather(rp, ip)

                @pl.when(j >= 1)
                def _():
                    gather_wait(1 - rp)
                    scatter_fire(1 - rp, (ip + 3) % 4)

                @pl.when(j + 2 < CHUNKS_PER_WORKER)
                def _():
                    idx_fetch(j + 2, (ip + 2) % 4)

        return carry

    lax.fori_loop(0, (CHUNKS_PER_WORKER + UNROLL - 1) // UNROLL, body, 0)
    scatter_wait()
    gather_wait((CHUNKS_PER_WORKER - 1) % 2)
    scatter_fire((CHUNKS_PER_WORKER - 1) % 2, (CHUNKS_PER_WORKER - 1) % 4)
    scatter_wait()
    plsc.subcore_barrier()
    out_d = [
        pltpu.async_copy(
            acc_sh.at[pl.ds(r0 + q * (ROWS_PER_TILE // 4), ROWS_PER_TILE // 4)],
            z_out.at[pl.ds(c * NPAD + r0 + q * (ROWS_PER_TILE // 4),
                           ROWS_PER_TILE // 4)],
            sem_s)
        for q in range(4)
    ]
    for d in out_d:
        d.wait()


_BLK = 2048
_GRID = NPAD // _BLK


def _dinv_of(deg_ref):
    return lax.rsqrt(deg_ref[0] + deg_ref[1] + 1.0)


def _tc_mm_body(x_ref, w_ref, y_ref):
    y_ref[...] = jnp.dot(x_ref[...], w_ref[...],
                         preferred_element_type=jnp.float32)


def _tc_scale_body(xw_ref, deg_ref, y_ref):
    y_ref[...] = xw_ref[...] * _dinv_of(deg_ref)


def _tc_mid_body(z_ref, y_ref, w_ref, b_ref, deg_ref, o_ref):
    dinv = _dinv_of(deg_ref)
    z = z_ref[0] + z_ref[1] - y_ref[...]
    h = jnp.maximum(z * dinv + b_ref[...], 0.0)
    o_ref[...] = (
        jnp.dot(h, w_ref[...], preferred_element_type=jnp.float32) * dinv
    )


def _tc3_body(z_ref, y_ref, w_ref, b_ref, blin_ref, deg_ref, o_ref):
    dinv = _dinv_of(deg_ref)
    z = z_ref[0] + z_ref[1] - y_ref[...]
    h = jnp.maximum(z * dinv + b_ref[...], 0.0)
    o_ref[...] = (
        jnp.dot(h, w_ref[...], preferred_element_type=jnp.float32)
        + blin_ref[...]
    )


_deg_spec = pl.BlockSpec((NC, _BLK, 1), lambda i: (0, i, 0))
_z_spec = pl.BlockSpec((NC, _BLK, D), lambda i: (0, i, 0))
_row_spec = pl.BlockSpec((_BLK, D), lambda i: (i, 0))

_tc_mm = pl.pallas_call(
    _tc_mm_body,
    grid=(N // 2000,),
    in_specs=[
        pl.BlockSpec((2000, D), lambda i: (i, 0)),
        pl.BlockSpec((D, D), lambda i: (0, 0)),
    ],
    out_specs=pl.BlockSpec((2000, D), lambda i: (i, 0)),
    out_shape=jax.ShapeDtypeStruct((NPAD, D), jnp.float32),
)

_tc_scale = pl.pallas_call(
    _tc_scale_body,
    grid=(_GRID,),
    in_specs=[_row_spec, _deg_spec],
    out_specs=_row_spec,
    out_shape=jax.ShapeDtypeStruct((NPAD, D), jnp.float32),
)

_tc_mid = pl.pallas_call(
    _tc_mid_body,
    grid=(_GRID,),
    in_specs=[
        _z_spec,
        _row_spec,
        pl.BlockSpec((D, D), lambda i: (0, 0)),
        pl.BlockSpec((1, D), lambda i: (0, 0)),
        _deg_spec,
    ],
    out_specs=_row_spec,
    out_shape=jax.ShapeDtypeStruct((NPAD, D), jnp.float32),
)

_tc3 = pl.pallas_call(
    _tc3_body,
    grid=(N // 2000,),
    in_specs=[
        pl.BlockSpec((NC, 2000, D), lambda i: (0, i, 0)),
        pl.BlockSpec((2000, D), lambda i: (i, 0)),
        pl.BlockSpec((D, 1), lambda i: (0, 0)),
        pl.BlockSpec((1, D), lambda i: (0, 0)),
        pl.BlockSpec((1, 1), lambda i: (0, 0)),
        pl.BlockSpec((NC, 2000, 1), lambda i: (0, i, 0)),
    ],
    out_specs=pl.BlockSpec((2000, 1), lambda i: (i, 0)),
    out_shape=jax.ShapeDtypeStruct((N, 1), jnp.float32),
)


def kernel(x, edge_index, W1, b1, W2, b2, Wlin, blin):
    npad_e = E_PAD - E
    pad_idx = (jnp.arange(npad_e, dtype=jnp.int32) % (NPAD - N)) + N
    e_all = jnp.concatenate(
        [edge_index.astype(jnp.int32),
         jnp.broadcast_to(pad_idx, (2, npad_e))], axis=1)
    eidx = jnp.transpose(e_all.reshape(2, -1, CHUNK), (1, 0, 2))

    deg = _deg_kernel(eidx)
    deg3 = deg.reshape(NC, NPAD, 1)
    xw1 = _tc_mm(x, W1)
    y1 = _tc_scale(xw1, deg3)
    z1 = _prop_kernel(y1, eidx).reshape(NC, NPAD, D)
    y2 = _tc_mid(z1, y1, W2, b1.reshape(1, D), deg3)
    z2 = _prop_kernel(y2, eidx).reshape(NC, NPAD, D)
    return _tc3(z2, y2, Wlin, b2.reshape(1, D), blin.reshape(1, 1), deg3)


# speedup vs baseline: 1.0348x; 1.0009x over previous
import functools

import jax
import jax.numpy as jnp
from jax import lax
from jax.experimental import pallas as pl
from jax.experimental.pallas import tpu as pltpu
from jax.experimental.pallas import tpu_sc as plsc

N = 10000
E = 320000
D = 128

NC = 2
NS = 16

NPAD = 10240
ROWS_PER_TILE = NPAD // NS
CHUNK = 128
CHUNKS_PER_WORKER = 80
E_PAD = NC * NS * CHUNKS_PER_WORKER * CHUNK

_mesh = plsc.VectorSubcoreMesh(
    core_axis_name="c", subcore_axis_name="s", num_cores=NC, num_subcores=NS
)


@functools.partial(
    pl.kernel,
    out_type=jax.ShapeDtypeStruct((NC, NPAD), jnp.float32),
    mesh=_mesh,
    scratch_types=[
        pltpu.VMEM((CHUNKS_PER_WORKER, 2, CHUNK), jnp.int32),
        pltpu.VMEM((CHUNK,), jnp.float32),
        pltpu.VMEM((ROWS_PER_TILE,), jnp.float32),
        pltpu.VMEM_SHARED((NPAD,), jnp.float32),
        pltpu.SemaphoreType.DMA,
    ],
)
def _deg_kernel(eidx_hbm, deg_out, idx_v, ones_v, zrow_v, hist_sh, sem_s):
    c = lax.axis_index("c")
    s = lax.axis_index("s")
    w = s * NC + c
    for i in range(CHUNK // 16):
        ones_v[pl.ds(i * 16, 16)] = jnp.full((16,), 1.0, jnp.float32)
    for i in range(ROWS_PER_TILE // 16):
        zrow_v[pl.ds(i * 16, 16)] = jnp.zeros((16,), jnp.float32)
    pltpu.sync_copy(zrow_v, hist_sh.at[pl.ds(s * ROWS_PER_TILE, ROWS_PER_TILE)])
    pltpu.sync_copy(eidx_hbm.at[pl.ds(w * CHUNKS_PER_WORKER, CHUNKS_PER_WORKER)],
                    idx_v)
    plsc.subcore_barrier()

    DEPTH = 8

    def body(j, carry):
        pltpu.async_copy(ones_v, hist_sh.at[idx_v.at[j, 1]], sem_s, add=True)

        @pl.when(j >= DEPTH)
        def _():
            pltpu.make_async_copy(ones_v, hist_sh.at[idx_v.at[j, 1]], sem_s).wait()

        return carry

    lax.fori_loop(0, CHUNKS_PER_WORKER, body, 0)
    for _ in range(DEPTH):
        pltpu.make_async_copy(ones_v, hist_sh.at[idx_v.at[0, 1]], sem_s).wait()
    plsc.subcore_barrier()
    pltpu.sync_copy(
        hist_sh.at[pl.ds(s * ROWS_PER_TILE, ROWS_PER_TILE)],
        deg_out.at[c, pl.ds(s * ROWS_PER_TILE, ROWS_PER_TILE)],
    )


@functools.partial(
    pl.kernel,
    out_type=jax.ShapeDtypeStruct((NC * NPAD, D), jnp.float32),
    mesh=_mesh,
    scratch_types=[
        pltpu.VMEM((2, CHUNK), jnp.int32),
        pltpu.VMEM((2, CHUNK), jnp.int32),
        pltpu.VMEM((2, CHUNK), jnp.int32),
        pltpu.VMEM((2, CHUNK), jnp.int32),
        pltpu.VMEM((CHUNK, D), jnp.float32),
        pltpu.VMEM((CHUNK, D), jnp.float32),
        pltpu.VMEM_SHARED((NPAD, D), jnp.float32),
        pltpu.SemaphoreType.DMA,
        pltpu.SemaphoreType.DMA,
        pltpu.SemaphoreType.DMA,
    ],
)
def _prop_kernel(y_hbm, eidx_hbm, z_out,
                 idx_a, idx_b, idx_c, idx_d, rows_a, rows_b, acc_sh,
                 sem_i, sem_g, sem_s):
    idx = [idx_a, idx_b, idx_c, idx_d]
    rows = [rows_a, rows_b]
    c = lax.axis_index("c")
    s = lax.axis_index("s")
    w = s * NC + c
    r0 = s * ROWS_PER_TILE

    init_d = [
        pltpu.async_copy(
            y_hbm.at[pl.ds(r0 + q * (ROWS_PER_TILE // 4), ROWS_PER_TILE // 4)],
            acc_sh.at[pl.ds(r0 + q * (ROWS_PER_TILE // 4), ROWS_PER_TILE // 4)],
            sem_s)
        for q in range(4)
    ]
    for d in init_d:
        d.wait()
    plsc.subcore_barrier()

    def idx_fetch(j, p):
        pltpu.async_copy(eidx_hbm.at[w * CHUNKS_PER_WORKER + j], idx[p], sem_i)

    def idx_wait(p):
        pltpu.make_async_copy(eidx_hbm.at[0], idx[p], sem_i).wait()

    def gather(rp, ip):
        pltpu.async_copy(y_hbm.at[idx[ip].at[0]], rows[rp], sem_g)

    def gather_wait(p):
        pltpu.make_async_copy(y_hbm.at[idx[0].at[0]], rows[p], sem_g).wait()

    def scatter_fire(rp, ip):
        pltpu.async_copy(rows[rp], acc_sh.at[idx[ip].at[1]], sem_s, add=True)

    def scatter_wait():
        pltpu.make_async_copy(rows[0], acc_sh.at[idx[0].at[1]], sem_s).wait()

    idx_fetch(0, 0)
    idx_fetch(1, 1)

    UNROLL = 4

    def body(g, carry):
        for u in range(UNROLL):
            j = g * UNROLL + u
            rp = u % 2
            ip = u % 4

            @pl.when(j < CHUNKS_PER_WORKER)
            def _():
                @pl.when(j >= 2)
                def _():
                    scatter_wait()

                idx_wait(ip)
                gather(rp, ip)

                @pl.when(j >= 1)
                def _():
                    gather_wait(1 - rp)
                    scatter_fire(1 - rp, (ip + 3) % 4)

                @pl.when(j + 2 < CHUNKS_PER_WORKER)
                def _():
                    idx_fetch(j + 2, (ip + 2) % 4)

        return carry

    lax.fori_loop(0, (CHUNKS_PER_WORKER + UNROLL - 1) // UNROLL, body, 0)
    scatter_wait()
    gather_wait((CHUNKS_PER_WORKER - 1) % 2)
    scatter_fire((CHUNKS_PER_WORKER - 1) % 2, (CHUNKS_PER_WORKER - 1) % 4)
    scatter_wait()
    plsc.subcore_barrier()
    out_d = [
        pltpu.async_copy(
            acc_sh.at[pl.ds(r0 + q * (ROWS_PER_TILE // 4), ROWS_PER_TILE // 4)],
            z_out.at[pl.ds(c * NPAD + r0 + q * (ROWS_PER_TILE // 4),
                           ROWS_PER_TILE // 4)],
            sem_s)
        for q in range(4)
    ]
    for d in out_d:
        d.wait()


_BLK = 2560
_GRID = NPAD // _BLK


def _dinv_of(deg_ref):
    return lax.rsqrt(deg_ref[0] + deg_ref[1] + 1.0)


def _tc_mm_body(x_ref, w_ref, y_ref):
    y_ref[...] = jnp.dot(x_ref[...], w_ref[...],
                         preferred_element_type=jnp.float32)


def _tc_scale_body(xw_ref, deg_ref, y_ref):
    y_ref[...] = xw_ref[...] * _dinv_of(deg_ref)


def _tc_mid_body(z_ref, y_ref, w_ref, b_ref, deg_ref, o_ref):
    dinv = _dinv_of(deg_ref)
    z = z_ref[0] + z_ref[1] - y_ref[...]
    h = jnp.maximum(z * dinv + b_ref[...], 0.0)
    o_ref[...] = (
        jnp.dot(h, w_ref[...], preferred_element_type=jnp.float32) * dinv
    )


def _tc3_body(z_ref, y_ref, w_ref, b_ref, blin_ref, deg_ref, o_ref):
    dinv = _dinv_of(deg_ref)
    z = z_ref[0] + z_ref[1] - y_ref[...]
    h = jnp.maximum(z * dinv + b_ref[...], 0.0)
    o_ref[...] = (
        jnp.dot(h, w_ref[...], preferred_element_type=jnp.float32)
        + blin_ref[...]
    )


_deg_spec = pl.BlockSpec((NC, _BLK, 1), lambda i: (0, i, 0))
_z_spec = pl.BlockSpec((NC, _BLK, D), lambda i: (0, i, 0))
_row_spec = pl.BlockSpec((_BLK, D), lambda i: (i, 0))

_tc_mm = pl.pallas_call(
    _tc_mm_body,
    grid=(N // 2000,),
    in_specs=[
        pl.BlockSpec((2000, D), lambda i: (i, 0)),
        pl.BlockSpec((D, D), lambda i: (0, 0)),
    ],
    out_specs=pl.BlockSpec((2000, D), lambda i: (i, 0)),
    out_shape=jax.ShapeDtypeStruct((NPAD, D), jnp.float32),
)

_tc_scale = pl.pallas_call(
    _tc_scale_body,
    grid=(_GRID,),
    in_specs=[_row_spec, _deg_spec],
    out_specs=_row_spec,
    out_shape=jax.ShapeDtypeStruct((NPAD, D), jnp.float32),
)

_tc_mid = pl.pallas_call(
    _tc_mid_body,
    grid=(_GRID,),
    in_specs=[
        _z_spec,
        _row_spec,
        pl.BlockSpec((D, D), lambda i: (0, 0)),
        pl.BlockSpec((1, D), lambda i: (0, 0)),
        _deg_spec,
    ],
    out_specs=_row_spec,
    out_shape=jax.ShapeDtypeStruct((NPAD, D), jnp.float32),
)

_tc3 = pl.pallas_call(
    _tc3_body,
    grid=(N // 2000,),
    in_specs=[
        pl.BlockSpec((NC, 2000, D), lambda i: (0, i, 0)),
        pl.BlockSpec((2000, D), lambda i: (i, 0)),
        pl.BlockSpec((D, 1), lambda i: (0, 0)),
        pl.BlockSpec((1, D), lambda i: (0, 0)),
        pl.BlockSpec((1, 1), lambda i: (0, 0)),
        pl.BlockSpec((NC, 2000, 1), lambda i: (0, i, 0)),
    ],
    out_specs=pl.BlockSpec((2000, 1), lambda i: (i, 0)),
    out_shape=jax.ShapeDtypeStruct((N, 1), jnp.float32),
)


def kernel(x, edge_index, W1, b1, W2, b2, Wlin, blin):
    npad_e = E_PAD - E
    pad_idx = (jnp.arange(npad_e, dtype=jnp.int32) % (NPAD - N)) + N
    e_all = jnp.concatenate(
        [edge_index.astype(jnp.int32),
         jnp.broadcast_to(pad_idx, (2, npad_e))], axis=1)
    eidx = jnp.transpose(e_all.reshape(2, -1, CHUNK), (1, 0, 2))

    deg = _deg_kernel(eidx)
    deg3 = deg.reshape(NC, NPAD, 1)
    xw1 = _tc_mm(x, W1)
    y1 = _tc_scale(xw1, deg3)
    z1 = _prop_kernel(y1, eidx).reshape(NC, NPAD, D)
    y2 = _tc_mid(z1, y1, W2, b1.reshape(1, D), deg3)
    z2 = _prop_kernel(y2, eidx).reshape(NC, NPAD, D)
    return _tc3(z2, y2, Wlin, b2.reshape(1, D), blin.reshape(1, 1), deg3)
